# baseline (speedup 1.0000x reference)
import jax
import jax.numpy as jnp
from jax.experimental import pallas as pl
from jax.experimental.pallas import tpu as pltpu

N = 4096
F = 64
BM = 512
NB = N // BM


def _sgconv_kernel(x_ref, adj_ref, out_ref, adjbf, h1bf):
    t = pl.program_id(0)

    @pl.when(t == 0)
    def _init():
        h1bf[...] = jnp.zeros((N, F), jnp.bfloat16)

    abf = adj_ref[...].astype(jnp.bfloat16)
    adjbf[pl.ds(t * BM, BM), :] = abf

    out_ref[pl.ds(t * BM, BM), :] = jnp.dot(
        abf, h1bf[...], preferred_element_type=jnp.float32)

    h1blk = jnp.dot(abf, x_ref[...], preferred_element_type=jnp.float32)
    h1bf[pl.ds(t * BM, BM), :] = h1blk.astype(jnp.bfloat16)

    out_ref[...] = out_ref[...] + jnp.dot(
        adjbf[:, pl.ds(t * BM, BM)], h1bf[pl.ds(t * BM, BM), :],
        preferred_element_type=jnp.float32)


@jax.jit
def kernel(x, adj):
    return pl.pallas_call(
        _sgconv_kernel,
        grid=(NB,),
        in_specs=[
            pl.BlockSpec((N, F), lambda t: (0, 0)),
            pl.BlockSpec((BM, N), lambda t: (t, 0)),
        ],
        out_specs=pl.BlockSpec((N, F), lambda t: (0, 0)),
        out_shape=jax.ShapeDtypeStruct((N, F), jnp.float32),
        scratch_shapes=[
            pltpu.VMEM((N, N), jnp.bfloat16),
            pltpu.VMEM((N, F), jnp.bfloat16),
        ],
    )(x.astype(jnp.bfloat16), adj)
